# baseline (speedup 1.0000x reference)
import jax
import jax.numpy as jnp
from jax.experimental import pallas as pl
from jax.experimental.pallas import tpu as pltpu

B, C, H, W = 32, 768, 32, 32
HW = H * W
CHUNK = 256
NBUF = 8
NSTEP = (B * C) // CHUNK
KC = C // CHUNK


def _body(m_hbm, f_hbm, t_hbm, o_hbm,
          mvmem, tvmem, fbuf, obuf,
          msem, tsem, insem, outsem):
    pltpu.make_async_copy(m_hbm, mvmem, msem).start()
    pltpu.make_async_copy(t_hbm, tvmem, tsem).start()
    for s in range(NBUF):
        pltpu.make_async_copy(
            f_hbm.at[pl.ds(s * CHUNK, CHUNK)], fbuf.at[s], insem.at[s]).start()
    pltpu.make_async_copy(m_hbm, mvmem, msem).wait()
    pltpu.make_async_copy(t_hbm, tvmem, tsem).wait()

    def step_fn(step, carry):
        slot = jax.lax.rem(step, NBUF)
        pltpu.make_async_copy(
            f_hbm.at[pl.ds(step * CHUNK, CHUNK)], fbuf.at[slot],
            insem.at[slot]).wait()

        @pl.when(step >= NBUF)
        def _():
            pltpu.make_async_copy(
                obuf.at[slot], o_hbm.at[pl.ds((step - NBUF) * CHUNK, CHUNK)],
                outsem.at[slot]).wait()

        b = step // KC
        c0 = jax.lax.rem(step, KC) * CHUNK
        m = mvmem[pl.ds(b, 1), :] != 0
        t = tvmem[pl.ds(c0, CHUNK), :]
        obuf[slot] = jnp.where(m, fbuf[slot], t)

        pltpu.make_async_copy(
            obuf.at[slot], o_hbm.at[pl.ds(step * CHUNK, CHUNK)],
            outsem.at[slot]).start()

        @pl.when(step + NBUF < NSTEP)
        def _():
            pltpu.make_async_copy(
                f_hbm.at[pl.ds((step + NBUF) * CHUNK, CHUNK)], fbuf.at[slot],
                insem.at[slot]).start()
        return carry

    jax.lax.fori_loop(0, NSTEP, step_fn, 0)

    for s in range(NBUF):
        step = NSTEP - NBUF + s
        slot = step % NBUF
        pltpu.make_async_copy(
            obuf.at[slot], o_hbm.at[pl.ds(step * CHUNK, CHUNK)],
            outsem.at[slot]).wait()


def kernel(features, active_mask, mask_token):
    f2 = features.reshape(B * C, HW)
    m2 = active_mask.astype(jnp.int32).reshape(B, HW)
    t2 = jnp.broadcast_to(mask_token.reshape(C, 1), (C, HW))
    out = pl.pallas_call(
        _body,
        in_specs=[
            pl.BlockSpec(memory_space=pl.ANY),
            pl.BlockSpec(memory_space=pl.ANY),
            pl.BlockSpec(memory_space=pl.ANY),
        ],
        out_specs=pl.BlockSpec(memory_space=pl.ANY),
        out_shape=jax.ShapeDtypeStruct((B * C, HW), jnp.float32),
        scratch_shapes=[
            pltpu.VMEM((B, HW), jnp.int32),
            pltpu.VMEM((C, HW), jnp.float32),
            pltpu.VMEM((NBUF, CHUNK, HW), jnp.float32),
            pltpu.VMEM((NBUF, CHUNK, HW), jnp.float32),
            pltpu.SemaphoreType.DMA,
            pltpu.SemaphoreType.DMA,
            pltpu.SemaphoreType.DMA((NBUF,)),
            pltpu.SemaphoreType.DMA((NBUF,)),
        ],
    )(m2, f2, t2)
    return out.reshape(B, C, H, W)
